# baseline (speedup 1.0000x reference)
import functools

import jax
import jax.numpy as jnp
import numpy as np
from jax import lax
from jax.experimental import pallas as pl
from jax.experimental.pallas import tpu as pltpu
from jax.experimental.pallas import tpu_sc as plsc

B, C, H, D, F, G = 8, 2048, 16, 64, 64, 1024
NPROB = B * H
NW = 32
PPW = NPROB // NW
NV = C // 16
NVG = G // 16
MININT = np.int32(-2**31)

_mesh = functools.partial(
    plsc.VectorSubcoreMesh, core_axis_name="c", subcore_axis_name="s")
_SC_PARAMS = pltpu.CompilerParams(needs_layout_passes=False)
_SC_PARAMS_LINEAR = pltpu.CompilerParams(
    needs_layout_passes=False, use_tc_tiling_on_sc=False)


def _wid():
    return lax.axis_index("s") * 2 + lax.axis_index("c")


def _sort_body(score_hbm, heap_hbm, src_hbm, dst_hbm, bot_hbm,
               score_v, keyA, idxA, keyB, idxB, hist, heapo, srco, dsto,
               boto, sem):
    lanes = lax.iota(jnp.int32, 16)
    ones = jnp.ones((16,), jnp.int32)
    wid = _wid()
    for pp in range(PPW):
        p = wid * PPW + pp
        b = p // H
        h = p % H
        pltpu.sync_copy(score_hbm.at[p], score_v)

        def build(v, _):
            e = lanes * NV + v
            s = plsc.load_gather(score_v, [e])
            bits = plsc.bitcast(s, jnp.int32)
            key = jnp.where(bits < 0, bits, ~(bits | MININT))
            keyA[pl.ds(v * 16, 16)] = key
            idxA[pl.ds(v * 16, 16)] = e
            return 0
        lax.fori_loop(0, NV, build, 0)

        bufs = [(keyA, idxA, keyB, idxB), (keyB, idxB, keyA, idxA)]
        for pno in range(6):
            kin, iin, kout, iout = bufs[pno % 2]
            shift = jnp.int32(6 * pno)

            def zero(d, _):
                hist[pl.ds(d * 16, 16)] = jnp.zeros((16,), jnp.int32)
                return 0
            lax.fori_loop(0, 64, zero, 0)

            def hgram(v, _):
                k = kin[pl.ds(v * 16, 16)]
                d = lax.shift_right_logical(k, shift) & 63
                plsc.addupdate_scatter(hist, [d * 16 + lanes], ones)
                return 0
            lax.fori_loop(0, NV, hgram, 0)

            def prefix(d, carry):
                hv = hist[pl.ds(d * 16, 16)]
                cs = plsc.cumsum(hv)
                hist[pl.ds(d * 16, 16)] = (cs - hv) + carry
                return carry + jnp.sum(hv)
            lax.fori_loop(0, 64, prefix, jnp.int32(0))

            last = pno == 5

            def permute(v, _):
                k = kin[pl.ds(v * 16, 16)]
                iv = iin[pl.ds(v * 16, 16)]
                d = lax.shift_right_logical(k, shift) & 63
                slot = d * 16 + lanes
                pos = plsc.load_gather(hist, [slot])
                plsc.addupdate_scatter(hist, [slot], ones)
                if last:
                    addr = pos
                else:
                    addr = (pos & (NV - 1)) * 16 + lax.shift_right_logical(
                        pos, 7)
                plsc.store_scatter(kout, [addr], k)
                plsc.store_scatter(iout, [addr], iv)
                return 0
            lax.fori_loop(0, NV, permute, 0)

        src_base = b * (C * H) + h
        dst_base = b * (G * H) + h

        def emit(v, _):
            k = keyA[pl.ds(v * 16, 16)]
            iv = idxA[pl.ds(v * 16, 16)]
            m = ~k
            rbits = jnp.where(m < 0, m & jnp.int32(0x7FFFFFFF), ~m)
            heapo[pl.ds(v * 16, 16)] = plsc.bitcast(rbits, jnp.float32)
            g = v * 16 + lanes
            row = v >> 3
            col = (v & 7) * 16
            srco[row, pl.ds(col, 16)] = iv * H + src_base
            dsto[row, pl.ds(col, 16)] = g * H + dst_base
            return 0
        lax.fori_loop(0, NVG, emit, 0)

        def emit_bot(v, _):
            iv = idxA[pl.ds(G + v * 16, 16)]
            row = v >> 3
            col = (v & 7) * 16
            boto[row, pl.ds(col, 16)] = iv * H + src_base
            return 0
        lax.fori_loop(0, (C - G) // 16, emit_bot, 0)

        pltpu.sync_copy(heapo, heap_hbm.at[p])
        pltpu.sync_copy(srco, src_hbm.at[p])
        pltpu.sync_copy(dsto, dst_hbm.at[p])
        pltpu.sync_copy(boto, bot_hbm.at[p])


def _sort_call(score_t):
    idx_t = jax.ShapeDtypeStruct((NPROB, G // 128, 128), jnp.int32)
    return pl.kernel(
        _sort_body,
        out_type=[
            jax.ShapeDtypeStruct((NPROB, G), jnp.float32),
            idx_t, idx_t,
            jax.ShapeDtypeStruct((NPROB, (C - G) // 128, 128), jnp.int32),
        ],
        mesh=_mesh(),
        scratch_types=[
            pltpu.VMEM((C,), jnp.float32),
            pltpu.VMEM((C,), jnp.int32),
            pltpu.VMEM((C,), jnp.int32),
            pltpu.VMEM((C,), jnp.int32),
            pltpu.VMEM((C,), jnp.int32),
            pltpu.VMEM((1024,), jnp.int32),
            pltpu.VMEM((G,), jnp.float32),
            pltpu.VMEM((G // 128, 128), jnp.int32),
            pltpu.VMEM((G // 128, 128), jnp.int32),
            pltpu.VMEM(((C - G) // 128, 128), jnp.int32),
            pltpu.SemaphoreType.DMA,
        ],
        compiler_params=_SC_PARAMS_LINEAR,
    )(score_t)


CB = C - G


def _gather_body(kf, vf, fkf, src_hbm, dst_hbm, bot_hbm,
                 ko, vo, fko, qb,
                 srcv, dstv, botv, buf, sem_g, sem_s):
    wid = _wid()
    nchunk = G // 128
    nbchunk = CB // 128
    for pp in range(PPW):
        p = wid * PPW + pp
        pltpu.sync_copy(src_hbm.at[p], srcv)
        pltpu.sync_copy(dst_hbm.at[p], dstv)
        pltpu.sync_copy(bot_hbm.at[p], botv)
        for tab, out in ((kf, ko), (vf, vo), (fkf, fko)):
            gathers = [
                pltpu.async_copy(tab.at[srcv.at[j]],
                                 buf.at[pl.ds(j * 128, 128)], sem_g)
                for j in range(nchunk)
            ]
            for cp in gathers:
                cp.wait()
            scatters = [
                pltpu.async_copy(buf.at[pl.ds(j * 128, 128)],
                                 out.at[dstv.at[j]], sem_s)
                for j in range(nchunk)
            ]
            for cp in scatters:
                cp.wait()
        for tab, lane0 in ((fkf, 0), (vf, D)):
            gathers = [
                pltpu.async_copy(tab.at[botv.at[j]],
                                 buf.at[pl.ds(j * 128, 128)], sem_g)
                for j in range(nbchunk)
            ]
            for cp in gathers:
                cp.wait()
            pltpu.sync_copy(buf.at[pl.ds(0, CB)],
                            qb.at[p, :, pl.ds(lane0, D)])


def _gather_call(kf, vf, fkf, src_idx, dst_idx, bot_idx):
    rows = jax.ShapeDtypeStruct((B * G * H, D), jnp.float32)
    botrows = jax.ShapeDtypeStruct((NPROB, CB, 128), jnp.float32)
    return pl.kernel(
        _gather_body,
        out_type=[rows, rows, rows, botrows],
        mesh=_mesh(),
        scratch_types=[
            pltpu.VMEM((G // 128, 128), jnp.int32),
            pltpu.VMEM((G // 128, 128), jnp.int32),
            pltpu.VMEM((CB // 128, 128), jnp.int32),
            pltpu.VMEM((G, D), jnp.float32),
            pltpu.SemaphoreType.DMA,
            pltpu.SemaphoreType.DMA,
        ],
        compiler_params=_SC_PARAMS_LINEAR,
    )(kf, vf, fkf, src_idx, dst_idx, bot_idx)


def _einsum_body(q_ref, h_ref, s_ref):
    a = q_ref[0, :, :D]
    b = q_ref[0, :, D:]
    h_ref[0] = jax.lax.dot_general(
        a, b, (((0,), (0,)), ((), ())),
        preferred_element_type=jnp.float32)
    s_ref[0, 0] = jnp.sum(a, axis=0)


def _pallas_einsum(qb):
    return pl.pallas_call(
        _einsum_body,
        grid=(NPROB,),
        in_specs=[
            pl.BlockSpec((1, CB, 128), lambda i: (i, 0, 0)),
        ],
        out_specs=[
            pl.BlockSpec((1, F, D), lambda i: (i, 0, 0)),
            pl.BlockSpec((1, 1, F), lambda i: (i, 0, 0)),
        ],
        out_shape=[
            jax.ShapeDtypeStruct((NPROB, F, D), jnp.float32),
            jax.ShapeDtypeStruct((NPROB, 1, F), jnp.float32),
        ],
    )(qb)


def kernel(k_c, v_c, fk_c, score_c):
    score_t = jnp.transpose(score_c, (0, 2, 1)).reshape(NPROB, C)
    heap_t, src_idx, dst_idx, bot_idx = _sort_call(score_t)

    kf = k_c.reshape(B * C * H, D)
    vf = v_c.reshape(B * C * H, D)
    fkf = fk_c.reshape(B * C * H, D)
    Kt, Vt, FKt, qb = _gather_call(kf, vf, fkf, src_idx, dst_idx, bot_idx)
    K_top = Kt.reshape(B, G, H, D)
    V_top = Vt.reshape(B, G, H, D)
    FK_top = FKt.reshape(B, G, H, F)
    Hs, Ss = _pallas_einsum(qb)
    H_sum = Hs.reshape(B, H, F, D)
    S_sum = Ss.reshape(B, H, F)

    heap_score = jnp.transpose(heap_t.reshape(B, H, G), (0, 2, 1))
    return (K_top, V_top, FK_top, heap_score, H_sum, S_sum)


# speedup vs baseline: 1.4847x; 1.4847x over previous
import functools

import jax
import jax.numpy as jnp
import numpy as np
from jax import lax
from jax.experimental import pallas as pl
from jax.experimental.pallas import tpu as pltpu
from jax.experimental.pallas import tpu_sc as plsc

B, C, H, D, F, G = 8, 2048, 16, 64, 64, 1024
NPROB = B * H
NW = 32
PPW = NPROB // NW
NV = C // 16
NVG = G // 16
MININT = np.int32(-2**31)
DC = 8

_mesh = functools.partial(
    plsc.VectorSubcoreMesh, core_axis_name="c", subcore_axis_name="s")
_SC_PARAMS = pltpu.CompilerParams(needs_layout_passes=False)


def _wid():
    return lax.axis_index("s") * 2 + lax.axis_index("c")


def _sort_body(score_hbm, heap_hbm, top_hbm, mask_hbm,
               score_v, keyA, idxA, keyB, idxB, hist, heapo, masko, sem):
    lanes = lax.iota(jnp.int32, 16)
    ones = jnp.ones((16,), jnp.int32)
    wid = _wid()
    for pp in range(PPW):
        p = wid * PPW + pp
        pltpu.sync_copy(score_hbm.at[p], score_v)

        def build(v, _):
            e = lanes * NV + v
            s = plsc.load_gather(score_v, [e])
            bits = plsc.bitcast(s, jnp.int32)
            key = jnp.where(bits < 0, bits, ~(bits | MININT))
            keyA[pl.ds(v * 16, 16)] = key
            idxA[pl.ds(v * 16, 16)] = e
            return 0
        lax.fori_loop(0, NV, build, 0)

        bufs = [(keyA, idxA, keyB, idxB), (keyB, idxB, keyA, idxA)]
        for pno in range(6):
            kin, iin, kout, iout = bufs[pno % 2]
            shift = jnp.int32(6 * pno)

            def zero(d, _):
                hist[pl.ds(d * 16, 16)] = jnp.zeros((16,), jnp.int32)
                return 0
            lax.fori_loop(0, 64, zero, 0)

            def hgram(v, _):
                k = kin[pl.ds(v * 16, 16)]
                d = lax.shift_right_logical(k, shift) & 63
                plsc.addupdate_scatter(hist, [d * 16 + lanes], ones)
                return 0
            lax.fori_loop(0, NV, hgram, 0)

            def prefix(d, carry):
                hv = hist[pl.ds(d * 16, 16)]
                cs = plsc.cumsum(hv)
                hist[pl.ds(d * 16, 16)] = (cs - hv) + carry
                return carry + jnp.sum(hv)
            lax.fori_loop(0, 64, prefix, jnp.int32(0))

            last = pno == 5

            def permute(v, _):
                k = kin[pl.ds(v * 16, 16)]
                iv = iin[pl.ds(v * 16, 16)]
                d = lax.shift_right_logical(k, shift) & 63
                slot = d * 16 + lanes
                pos = plsc.load_gather(hist, [slot])
                plsc.addupdate_scatter(hist, [slot], ones)
                if last:
                    addr = pos
                else:
                    addr = (pos & (NV - 1)) * 16 + lax.shift_right_logical(
                        pos, 7)
                plsc.store_scatter(kout, [addr], k)
                plsc.store_scatter(iout, [addr], iv)
                return 0
            lax.fori_loop(0, NV, permute, 0)

        def emit(v, _):
            k = keyA[pl.ds(v * 16, 16)]
            m = ~k
            rbits = jnp.where(m < 0, m & jnp.int32(0x7FFFFFFF), ~m)
            heapo[pl.ds(v * 16, 16)] = plsc.bitcast(rbits, jnp.float32)
            return 0
        lax.fori_loop(0, NVG, emit, 0)

        def mones(v, _):
            masko[pl.ds(v * 16, 16)] = jnp.ones((16,), jnp.float32)
            return 0
        lax.fori_loop(0, NV, mones, 0)

        zf = jnp.zeros((16,), jnp.float32)

        def mzero(v, _):
            iv = idxA[pl.ds(v * 16, 16)]
            plsc.store_scatter(masko, [iv], zf)
            return 0
        lax.fori_loop(0, NVG, mzero, 0)

        pltpu.sync_copy(heapo, heap_hbm.at[p])
        pltpu.sync_copy(idxA.at[pl.ds(0, G)], top_hbm.at[p])
        pltpu.sync_copy(masko, mask_hbm.at[p])


def _sort_call(score_t):
    return pl.kernel(
        _sort_body,
        out_type=[
            jax.ShapeDtypeStruct((NPROB, G), jnp.float32),
            jax.ShapeDtypeStruct((NPROB, G), jnp.int32),
            jax.ShapeDtypeStruct((NPROB, C), jnp.float32),
        ],
        mesh=_mesh(),
        scratch_types=[
            pltpu.VMEM((C,), jnp.float32),
            pltpu.VMEM((C,), jnp.int32),
            pltpu.VMEM((C,), jnp.int32),
            pltpu.VMEM((C,), jnp.int32),
            pltpu.VMEM((C,), jnp.int32),
            pltpu.VMEM((1024,), jnp.int32),
            pltpu.VMEM((G,), jnp.float32),
            pltpu.VMEM((C,), jnp.float32),
            pltpu.SemaphoreType.DMA,
        ],
        compiler_params=_SC_PARAMS,
    )(score_t)


def _gather_body(kt, vt, fkt, top_hbm, ko, vo, fko,
                 idxv, slabA, slabB, outs, sem_in, sem_out):
    wid = _wid()

    def per_problem(pp, _):
        p = wid * PPW + pp
        b = p // H
        h = p % H
        pltpu.sync_copy(top_hbm.at[p], idxv)
        for tab, out in ((kt, ko), (vt, vo), (fkt, fko)):
            def chunk(dc, _):
                pltpu.sync_copy(tab.at[b, h, pl.ds(dc * DC, DC)], slabA)

                def g(j, _):
                    ic = idxv[pl.ds(j * 16, 16)]
                    for d in range(DC):
                        row = jnp.full((16,), d, jnp.int32)
                        outs[d, pl.ds(j * 16, 16)] = plsc.load_gather(
                            slabA, [row, ic])
                    return 0
                lax.fori_loop(0, G // 16, g, 0)
                pltpu.sync_copy(outs, out.at[b, h, pl.ds(dc * DC, DC)])
                return 0
            lax.fori_loop(0, D // DC, chunk, 0)
        return 0
    lax.fori_loop(0, PPW, per_problem, 0)


def _gather_call(kt, vt, fkt, top_idx):
    out = jax.ShapeDtypeStruct((B, H, D, G), jnp.float32)
    return pl.kernel(
        _gather_body,
        out_type=[out, out, out],
        mesh=_mesh(),
        scratch_types=[
            pltpu.VMEM((G,), jnp.int32),
            pltpu.VMEM((DC, C), jnp.float32),
            pltpu.VMEM((DC, C), jnp.float32),
            pltpu.VMEM((DC, G), jnp.float32),
            pltpu.SemaphoreType.DMA,
            pltpu.SemaphoreType.DMA,
        ],
        compiler_params=_SC_PARAMS,
    )(kt, vt, fkt, top_idx)


def _einsum_body(fk_ref, v_ref, m_ref, h_ref, s_ref):
    m = m_ref[0, 0, 0]
    a = fk_ref[0, 0] * m[None, :]
    h_ref[0, 0] = jax.lax.dot_general(
        a, v_ref[0, 0], (((1,), (1,)), ((), ())),
        preferred_element_type=jnp.float32)
    s_ref[0, 0, 0] = jnp.sum(a, axis=1)


def _pallas_einsum(fkt, vt, mask4):
    return pl.pallas_call(
        _einsum_body,
        grid=(B, H),
        in_specs=[
            pl.BlockSpec((1, 1, F, C), lambda i, j: (i, j, 0, 0)),
            pl.BlockSpec((1, 1, D, C), lambda i, j: (i, j, 0, 0)),
            pl.BlockSpec((1, 1, 1, C), lambda i, j: (i, j, 0, 0)),
        ],
        out_specs=[
            pl.BlockSpec((1, 1, F, D), lambda i, j: (i, j, 0, 0)),
            pl.BlockSpec((1, 1, 1, F), lambda i, j: (i, j, 0, 0)),
        ],
        out_shape=[
            jax.ShapeDtypeStruct((B, H, F, D), jnp.float32),
            jax.ShapeDtypeStruct((B, H, 1, F), jnp.float32),
        ],
    )(fkt, vt, mask4)


def kernel(k_c, v_c, fk_c, score_c):
    score_t = jnp.transpose(score_c, (0, 2, 1)).reshape(NPROB, C)
    heap_t, top_idx, mask = _sort_call(score_t)

    kt = jnp.transpose(k_c, (0, 2, 3, 1))
    vt = jnp.transpose(v_c, (0, 2, 3, 1))
    fkt = jnp.transpose(fk_c, (0, 2, 3, 1))
    kg, vg, fkg = _gather_call(kt, vt, fkt, top_idx)
    K_top = jnp.transpose(kg, (0, 3, 1, 2))
    V_top = jnp.transpose(vg, (0, 3, 1, 2))
    FK_top = jnp.transpose(fkg, (0, 3, 1, 2))

    mask4 = mask.reshape(B, H, 1, C)
    Hs, Ss = _pallas_einsum(fkt, vt, mask4)
    H_sum = Hs
    S_sum = Ss.reshape(B, H, F)

    heap_score = jnp.transpose(heap_t.reshape(B, H, G), (0, 2, 1))
    return (K_top, V_top, FK_top, heap_score, H_sum, S_sum)


# speedup vs baseline: 1.9941x; 1.3431x over previous
import functools

import jax
import jax.numpy as jnp
import numpy as np
from jax import lax
from jax.experimental import pallas as pl
from jax.experimental.pallas import tpu as pltpu
from jax.experimental.pallas import tpu_sc as plsc

B, C, H, D, F, G = 8, 2048, 16, 64, 64, 1024
NPROB = B * H
NW = 32
PPW = NPROB // NW
NV = C // 16
NVG = G // 16
MININT = np.int32(-2**31)
DC = 8

_mesh = functools.partial(
    plsc.VectorSubcoreMesh, core_axis_name="c", subcore_axis_name="s")
_SC_PARAMS = pltpu.CompilerParams(needs_layout_passes=False)


def _wid():
    return lax.axis_index("s") * 2 + lax.axis_index("c")


def _sort_body(score_hbm, heap_hbm, top_hbm, mask_hbm,
               score_v, keyA, idxA, keyB, idxB, hist, heapo, masko, sem):
    lanes = lax.iota(jnp.int32, 16)
    ones = jnp.ones((16,), jnp.int32)
    wid = _wid()
    for pp in range(PPW):
        p = wid * PPW + pp
        pltpu.sync_copy(score_hbm.at[p], score_v)

        def build(v, _):
            e = lanes * NV + v
            s = plsc.load_gather(score_v, [e])
            bits = plsc.bitcast(s, jnp.int32)
            key = jnp.where(bits < 0, bits, ~(bits | MININT))
            keyA[pl.ds(v * 16, 16)] = key
            idxA[pl.ds(v * 16, 16)] = e
            return 0
        lax.fori_loop(0, NV, build, 0)

        bufs = [(keyA, idxA, keyB, idxB), (keyB, idxB, keyA, idxA)]
        for pno in range(6):
            kin, iin, kout, iout = bufs[pno % 2]
            shift = jnp.int32(6 * pno)

            def zero(d, _):
                hist[pl.ds(d * 16, 16)] = jnp.zeros((16,), jnp.int32)
                return 0
            lax.fori_loop(0, 64, zero, 0)

            def hgram(v, _):
                k = kin[pl.ds(v * 16, 16)]
                d = lax.shift_right_logical(k, shift) & 63
                plsc.addupdate_scatter(hist, [d * 16 + lanes], ones)
                return 0
            lax.fori_loop(0, NV, hgram, 0)

            def prefix(d, carry):
                hv = hist[pl.ds(d * 16, 16)]
                cs = plsc.cumsum(hv)
                hist[pl.ds(d * 16, 16)] = (cs - hv) + carry
                return carry + jnp.sum(hv)
            lax.fori_loop(0, 64, prefix, jnp.int32(0))

            last = pno == 5

            def permute(v, _):
                k = kin[pl.ds(v * 16, 16)]
                iv = iin[pl.ds(v * 16, 16)]
                d = lax.shift_right_logical(k, shift) & 63
                slot = d * 16 + lanes
                pos = plsc.load_gather(hist, [slot])
                plsc.addupdate_scatter(hist, [slot], ones)
                if last:
                    addr = pos
                else:
                    addr = (pos & (NV - 1)) * 16 + lax.shift_right_logical(
                        pos, 7)
                plsc.store_scatter(kout, [addr], k)
                plsc.store_scatter(iout, [addr], iv)
                return 0
            lax.fori_loop(0, NV, permute, 0)

        def emit(v, _):
            k = keyA[pl.ds(v * 16, 16)]
            m = ~k
            rbits = jnp.where(m < 0, m & jnp.int32(0x7FFFFFFF), ~m)
            heapo[pl.ds(v * 16, 16)] = plsc.bitcast(rbits, jnp.float32)
            return 0
        lax.fori_loop(0, NVG, emit, 0)

        def mones(v, _):
            masko[pl.ds(v * 16, 16)] = jnp.ones((16,), jnp.float32)
            return 0
        lax.fori_loop(0, NV, mones, 0)

        zf = jnp.zeros((16,), jnp.float32)

        def mzero(v, _):
            iv = idxA[pl.ds(v * 16, 16)]
            plsc.store_scatter(masko, [iv], zf)
            return 0
        lax.fori_loop(0, NVG, mzero, 0)

        pltpu.sync_copy(heapo, heap_hbm.at[p])
        pltpu.sync_copy(idxA.at[pl.ds(0, G)], top_hbm.at[p])
        pltpu.sync_copy(masko, mask_hbm.at[p])


def _sort_call(score_t):
    return pl.kernel(
        _sort_body,
        out_type=[
            jax.ShapeDtypeStruct((NPROB, G), jnp.float32),
            jax.ShapeDtypeStruct((NPROB, G), jnp.int32),
            jax.ShapeDtypeStruct((NPROB, C), jnp.float32),
        ],
        mesh=_mesh(),
        scratch_types=[
            pltpu.VMEM((C,), jnp.float32),
            pltpu.VMEM((C,), jnp.int32),
            pltpu.VMEM((C,), jnp.int32),
            pltpu.VMEM((C,), jnp.int32),
            pltpu.VMEM((C,), jnp.int32),
            pltpu.VMEM((1024,), jnp.int32),
            pltpu.VMEM((G,), jnp.float32),
            pltpu.VMEM((C,), jnp.float32),
            pltpu.SemaphoreType.DMA,
        ],
        compiler_params=_SC_PARAMS,
    )(score_t)


def _gather_body(kt, vt, fkt, top_hbm, ko, vo, fko,
                 idxv, slabA, slabB, outsA, outsB, sem_in, sem_out):
    wid = _wid()

    slabs = (slabA, slabB)
    outbufs = (outsA, outsB)

    def per_problem(pp, _):
        p = wid * PPW + pp
        b = p // H
        h = p % H
        pltpu.sync_copy(top_hbm.at[p], idxv)
        for tab, out in ((kt, ko), (vt, vo), (fkt, fko)):
            cin = pltpu.async_copy(tab.at[b, h, pl.ds(0, DC)], slabs[0],
                                   sem_in)
            pend = [None, None]
            for dc in range(D // DC):
                cin.wait()
                if dc + 1 < D // DC:
                    nxt = pltpu.async_copy(
                        tab.at[b, h, pl.ds((dc + 1) * DC, DC)],
                        slabs[(dc + 1) % 2], sem_in)
                slab = slabs[dc % 2]
                outb = outbufs[dc % 2]
                if pend[dc % 2] is not None:
                    pend[dc % 2].wait()

                def g(j, _):
                    ic = idxv[pl.ds(j * 16, 16)]
                    for d in range(DC):
                        row = jnp.full((16,), d, jnp.int32)
                        outb[d, pl.ds(j * 16, 16)] = plsc.load_gather(
                            slab, [row, ic])
                    return 0
                lax.fori_loop(0, G // 16, g, 0)
                pend[dc % 2] = pltpu.async_copy(
                    outb, out.at[b, h, pl.ds(dc * DC, DC)], sem_out)
                if dc + 1 < D // DC:
                    cin = nxt
            for q in pend:
                q.wait()
        return 0
    lax.fori_loop(0, PPW, per_problem, 0)


def _gather_call(kt, vt, fkt, top_idx):
    out = jax.ShapeDtypeStruct((B, H, D, G), jnp.float32)
    return pl.kernel(
        _gather_body,
        out_type=[out, out, out],
        mesh=_mesh(),
        scratch_types=[
            pltpu.VMEM((G,), jnp.int32),
            pltpu.VMEM((DC, C), jnp.float32),
            pltpu.VMEM((DC, C), jnp.float32),
            pltpu.VMEM((DC, G), jnp.float32),
            pltpu.VMEM((DC, G), jnp.float32),
            pltpu.SemaphoreType.DMA,
            pltpu.SemaphoreType.DMA,
        ],
        compiler_params=_SC_PARAMS,
    )(kt, vt, fkt, top_idx)


def _einsum_body(fk_ref, v_ref, m_ref, h_ref, s_ref):
    for h in range(H):
        m = m_ref[0, h, 0]
        a = fk_ref[0, h] * m[None, :]
        h_ref[0, h] = jax.lax.dot_general(
            a, v_ref[0, h], (((1,), (1,)), ((), ())),
            preferred_element_type=jnp.float32)
        s_ref[0, h, 0] = jnp.sum(a, axis=1)


def _pallas_einsum(fkt, vt, mask4):
    return pl.pallas_call(
        _einsum_body,
        grid=(B,),
        in_specs=[
            pl.BlockSpec((1, H, F, C), lambda i: (i, 0, 0, 0)),
            pl.BlockSpec((1, H, D, C), lambda i: (i, 0, 0, 0)),
            pl.BlockSpec((1, H, 1, C), lambda i: (i, 0, 0, 0)),
        ],
        out_specs=[
            pl.BlockSpec((1, H, F, D), lambda i: (i, 0, 0, 0)),
            pl.BlockSpec((1, H, 1, F), lambda i: (i, 0, 0, 0)),
        ],
        out_shape=[
            jax.ShapeDtypeStruct((B, H, F, D), jnp.float32),
            jax.ShapeDtypeStruct((B, H, 1, F), jnp.float32),
        ],
        compiler_params=pltpu.CompilerParams(
            vmem_limit_bytes=100 * 1024 * 1024),
    )(fkt, vt, mask4)


def kernel(k_c, v_c, fk_c, score_c):
    score_t = jnp.transpose(score_c, (0, 2, 1)).reshape(NPROB, C)
    heap_t, top_idx, mask = _sort_call(score_t)

    kt = jnp.transpose(k_c, (0, 2, 3, 1))
    vt = jnp.transpose(v_c, (0, 2, 3, 1))
    fkt = jnp.transpose(fk_c, (0, 2, 3, 1))
    kg, vg, fkg = _gather_call(kt, vt, fkt, top_idx)
    K_top = jnp.transpose(kg, (0, 3, 1, 2))
    V_top = jnp.transpose(vg, (0, 3, 1, 2))
    FK_top = jnp.transpose(fkg, (0, 3, 1, 2))

    mask4 = mask.reshape(B, H, 1, C)
    Hs, Ss = _pallas_einsum(fkt, vt, mask4)
    H_sum = Hs
    S_sum = Ss.reshape(B, H, F)

    heap_score = jnp.transpose(heap_t.reshape(B, H, G), (0, 2, 1))
    return (K_top, V_top, FK_top, heap_score, H_sum, S_sum)
